# baseline (speedup 1.0000x reference)
import dataclasses
import functools

import jax
import jax.numpy as jnp
from jax import lax
from jax.experimental import pallas as pl
from jax.experimental.pallas import tpu as pltpu
from jax.experimental.pallas import tpu_sc as plsc

N_NODES = 10000
D_FEAT = 128
HIDDEN = 32
NUM_LAYERS = 3
N_EDGES = 320000
NUM_GRAPHS = 64
EMB_DIM = HIDDEN * NUM_LAYERS

NC = 2
NS = 16
NW = NC * NS
LANES = 16

PAD_NODES = 10112
CW = 4
TAB = PAD_NODES * CW
ECH = 4096
EPAD = 327680

POOL_RPT = 320
POOL_PAD = NW * POOL_RPT
POOL_TAB = (NUM_GRAPHS + 1) * EMB_DIM

_sc_mesh = plsc.VectorSubcoreMesh(core_axis_name="c", subcore_axis_name="s")

_sc_params = pltpu.CompilerParams()
if "needs_layout_passes" in pltpu.CompilerParams.__dataclass_fields__:
    _sc_params = dataclasses.replace(_sc_params, needs_layout_passes=False)

_tc_params = pltpu.CompilerParams(vmem_limit_bytes=60 * 1024 * 1024)


def _make_agg_body(cgn, egn):
    ept = EPAD // egn
    nch = ept // ECH
    unroll = 8

    def body(hp_hbm, src_hbm, dst_hbm, out_hbm, utab_v, acc_v, sidx_v,
             didx_v, sem_s0, sem_s1, sem_d0, sem_d1):
        c = lax.axis_index("c")
        s = lax.axis_index("s")
        w = c * NS + s
        cg = w % cgn
        eg = w // cgn
        sem_s = [sem_s0, sem_s1]
        sem_d = [sem_d0, sem_d1]

        pltpu.sync_copy(hp_hbm.at[cg], utab_v)

        seed = jnp.where(eg == 0, jnp.float32(1.0), jnp.float32(0.0))

        @pl.loop(0, TAB // LANES)
        def _(i):
            sl = pl.ds(i * LANES, LANES)
            acc_v[sl] = utab_v[sl] * seed

        planes_u = [utab_v.at[pl.ds(j * PAD_NODES, PAD_NODES)]
                    for j in range(CW)]
        planes_a = [acc_v.at[pl.ds(j * PAD_NODES, PAD_NODES)]
                    for j in range(CW)]

        def copies(k, b):
            base = eg * ept + k * ECH
            return (pltpu.make_async_copy(src_hbm.at[pl.ds(base, ECH)],
                                          sidx_v.at[b], sem_s[b]),
                    pltpu.make_async_copy(dst_hbm.at[pl.ds(base, ECH)],
                                          didx_v.at[b], sem_d[b]))

        def issue(k, b):
            for cp in copies(k, b):
                cp.start()

        def wait(k, b):
            for cp in copies(k, b):
                cp.wait()

        def compute(b):
            @plsc.parallel_loop(0, ECH // LANES, unroll=unroll)
            def _(i):
                sl = pl.ds(i * LANES, LANES)
                sv = sidx_v[b, sl]
                dv = didx_v[b, sl]
                for j in range(CW):
                    vals = plsc.load_gather(planes_u[j], [sv])
                    plsc.addupdate_scatter(planes_a[j], [dv], vals)

        issue(0, 0)

        @pl.loop(0, nch, step=2)
        def _(k):
            issue(k + 1, 1)
            wait(k, 0)
            compute(0)

            @pl.when(k + 2 < nch)
            def _():
                issue(k + 2, 0)

            wait(k + 1, 1)
            compute(1)

        pltpu.sync_copy(acc_v, out_hbm.at[w])

    return body


def _make_sc_agg(cgn, egn):
    return pl.kernel(
        _make_agg_body(cgn, egn),
        out_type=jax.ShapeDtypeStruct((NW, TAB), jnp.float32),
        mesh=_sc_mesh,
        scratch_types=[
            pltpu.VMEM((TAB,), jnp.float32),
            pltpu.VMEM((TAB,), jnp.float32),
            pltpu.VMEM((2, ECH), jnp.int32),
            pltpu.VMEM((2, ECH), jnp.int32),
            pltpu.SemaphoreType.DMA,
            pltpu.SemaphoreType.DMA,
            pltpu.SemaphoreType.DMA,
            pltpu.SemaphoreType.DMA,
        ],
        compiler_params=_sc_params,
    )


@jax.jit
def _sc_agg_l1(h_pt, src, dst):
    return _make_sc_agg(32, 1)(h_pt, src, dst)


@jax.jit
def _sc_agg_l23(h_pt, src, dst):
    return _make_sc_agg(8, 4)(h_pt, src, dst)


def _to_planes(h, cgn):
    return h.T.reshape(cgn, CW, PAD_NODES).reshape(cgn, TAB)


def _from_partials(out, cgn, egn):
    return (out.reshape(egn, cgn, CW, PAD_NODES)
            .transpose(0, 3, 1, 2)
            .reshape(egn, PAD_NODES, cgn * CW))


def _pool_body(m_hbm, b_hbm, out_hbm, rows_v, bid_v, pacc_v):
    c = lax.axis_index("c")
    s = lax.axis_index("s")
    w = c * NS + s
    base = w * POOL_RPT

    pltpu.sync_copy(m_hbm.at[pl.ds(base, POOL_RPT)], rows_v)
    pltpu.sync_copy(b_hbm.at[pl.ds(base, POOL_RPT)], bid_v)

    @pl.loop(0, POOL_TAB // LANES)
    def _(i):
        pacc_v[pl.ds(i * LANES, LANES)] = jnp.zeros((LANES,), jnp.float32)

    colofs = lax.broadcasted_iota(jnp.int32, (LANES,), 0)

    @pl.loop(0, POOL_RPT // LANES)
    def _(g):
        for j in range(LANES):
            r = g * LANES + j
            rep = plsc.load_gather(
                bid_v, [jnp.zeros((LANES,), jnp.int32) + r])
            gbase = rep * EMB_DIM
            for k in range(EMB_DIM // LANES):
                vals = rows_v[r, pl.ds(k * LANES, LANES)]
                plsc.addupdate_scatter(
                    pacc_v, [gbase + (colofs + k * LANES)], vals)

    pltpu.sync_copy(pacc_v, out_hbm.at[w])


@jax.jit
def _sc_pool(m_p, batch_p):
    kern = pl.kernel(
        _pool_body,
        out_type=jax.ShapeDtypeStruct((NW, POOL_TAB), jnp.float32),
        mesh=_sc_mesh,
        scratch_types=[
            pltpu.VMEM((POOL_RPT, EMB_DIM), jnp.float32),
            pltpu.VMEM((POOL_RPT,), jnp.int32),
            pltpu.VMEM((POOL_TAB,), jnp.float32),
        ],
        compiler_params=_sc_params,
    )
    return kern(m_p, batch_p)


def _layer_kernel(m_ref, w1_ref, b1_ref, w2_ref, b2_ref, g_ref, bb_ref,
                  h_ref):
    m = m_ref[0]
    for e in range(1, m_ref.shape[0]):
        m = m + m_ref[e]
    pre = jnp.dot(m, w1_ref[...], preferred_element_type=jnp.float32) \
        + b1_ref[...]
    t = jnp.maximum(pre, 0.0)
    mm = jnp.dot(t, w2_ref[...], preferred_element_type=jnp.float32)
    mm = jnp.maximum(mm + b2_ref[...], 0.0)
    valid = lax.broadcasted_iota(jnp.int32, mm.shape, 0) < N_NODES
    mm = jnp.where(valid, mm, 0.0)
    mean = jnp.sum(mm, axis=0, keepdims=True) * (1.0 / N_NODES)
    d = jnp.where(valid, mm - mean, 0.0)
    var = jnp.sum(d * d, axis=0, keepdims=True) * (1.0 / N_NODES)
    h = d * jax.lax.rsqrt(var + 1e-5) * g_ref[...] + bb_ref[...]
    h_ref[...] = jnp.where(valid, h, 0.0)


@jax.jit
def _tc_layer(m_parts, w1, b1, w2, b2, g, bb):
    n = m_parts.shape[1]
    return pl.pallas_call(
        _layer_kernel,
        out_shape=jax.ShapeDtypeStruct((n, HIDDEN), jnp.float32),
        compiler_params=_tc_params,
    )(m_parts, w1, b1, w2, b2, g, bb)


def _ff_block(h, w1, b1, w2, b2, w3, b3, ws, bs):
    z = jnp.maximum(jnp.dot(h, w1, preferred_element_type=jnp.float32) + b1,
                    0.0)
    z = jnp.maximum(jnp.dot(z, w2, preferred_element_type=jnp.float32) + b2,
                    0.0)
    z = jnp.maximum(jnp.dot(z, w3, preferred_element_type=jnp.float32) + b3,
                    0.0)
    return z + jnp.dot(h, ws, preferred_element_type=jnp.float32) + bs


def _heads_kernel(m_ref, p_ref,
                  lw1, lb1, lw2, lb2, lw3, lb3, lws, lbs,
                  gw1, gb1, gw2, gb2, gw3, gb3, gws, gbs,
                  cw_ref, cb_ref, cl_ref,
                  z_ref, q_ref, g_ref, l_ref):
    y = jnp.sum(p_ref[...], axis=0)[:NUM_GRAPHS]
    g_ref[...] = _ff_block(y, gw1[...], gb1[...], gw2[...], gb2[...],
                           gw3[...], gb3[...], gws[...], gbs[...])
    l_ref[...] = _ff_block(m_ref[...], lw1[...], lb1[...], lw2[...], lb2[...],
                           lw3[...], lb3[...], lws[...], lbs[...])
    z = jnp.dot(y, cw_ref[...], preferred_element_type=jnp.float32) \
        + cb_ref[...]
    z_ref[...] = z
    cl = cl_ref[...]
    diff = z[:, None, :] - cl[None, :, :]
    dist = jnp.sum(diff * diff, axis=2)
    q = 1.0 / (1.0 + dist)
    q_ref[...] = q / jnp.sum(q, axis=1, keepdims=True)


@jax.jit
def _tc_heads(m_p, pooled, ld, gd, cw, cb, cl):
    n = m_p.shape[0]
    return pl.pallas_call(
        _heads_kernel,
        out_shape=(jax.ShapeDtypeStruct((NUM_GRAPHS, HIDDEN), jnp.float32),
                   jax.ShapeDtypeStruct((NUM_GRAPHS, HIDDEN), jnp.float32),
                   jax.ShapeDtypeStruct((NUM_GRAPHS, EMB_DIM), jnp.float32),
                   jax.ShapeDtypeStruct((n, EMB_DIM), jnp.float32)),
        compiler_params=_tc_params,
    )(m_p, pooled,
      ld['W1'], ld['b1'].reshape(1, -1), ld['W2'], ld['b2'].reshape(1, -1),
      ld['W3'], ld['b3'].reshape(1, -1), ld['Ws'], ld['bs'].reshape(1, -1),
      gd['W1'], gd['b1'].reshape(1, -1), gd['W2'], gd['b2'].reshape(1, -1),
      gd['W3'], gd['b3'].reshape(1, -1), gd['Ws'], gd['bs'].reshape(1, -1),
      cw, cb.reshape(1, -1), cl)


def kernel(x, edge_index, batch, num_graphs, params):
    del num_graphs
    pad_e = EPAD - N_EDGES
    src = jnp.concatenate(
        [edge_index[0], jnp.full((pad_e,), N_NODES, jnp.int32)])
    dst = jnp.concatenate(
        [edge_index[1], jnp.full((pad_e,), N_NODES, jnp.int32)])
    x_p = jnp.pad(x, ((0, PAD_NODES - N_NODES), (0, 0)))
    batch_p = jnp.concatenate(
        [batch, jnp.full((POOL_PAD - N_NODES,), NUM_GRAPHS, jnp.int32)])

    gin = params['gin']
    h = x_p
    hs = []
    for l in range(NUM_LAYERS):
        cgn, egn = (32, 1) if l == 0 else (8, 4)
        agg_fn = _sc_agg_l1 if l == 0 else _sc_agg_l23
        out = agg_fn(_to_planes(h, cgn), src, dst)
        m_parts = _from_partials(out, cgn, egn)
        p = gin[l]
        h = _tc_layer(m_parts, p['W1'], p['b1'].reshape(1, -1), p['W2'],
                      p['b2'].reshape(1, -1), p['bn_g'].reshape(1, -1),
                      p['bn_b'].reshape(1, -1))
        hs.append(h)

    m_p = jnp.pad(jnp.concatenate([h[:N_NODES] for h in hs], axis=1),
                  ((0, POOL_PAD - N_NODES), (0, 0)))
    pooled = _sc_pool(m_p, batch_p).reshape(NW, NUM_GRAPHS + 1, EMB_DIM)
    z, q, g_enc, l_enc_p = _tc_heads(
        m_p, pooled, params['local_d'], params['global_d'],
        params['cluster_W'], params['cluster_b'], params['cluster_layer'])
    return z, q, g_enc, l_enc_p[:N_NODES]


# speedup vs baseline: 1.0110x; 1.0110x over previous
import dataclasses
import functools

import jax
import jax.numpy as jnp
from jax import lax
from jax.experimental import pallas as pl
from jax.experimental.pallas import tpu as pltpu
from jax.experimental.pallas import tpu_sc as plsc

N_NODES = 10000
D_FEAT = 128
HIDDEN = 32
NUM_LAYERS = 3
N_EDGES = 320000
NUM_GRAPHS = 64
EMB_DIM = HIDDEN * NUM_LAYERS

NC = 2
NS = 16
NW = NC * NS
LANES = 16

PAD_NODES = 10112
CW = 4
TAB = PAD_NODES * CW
ECH = 4096
EPAD = 327680

POOL_RPT = 320
POOL_PAD = NW * POOL_RPT
POOL_TAB = (NUM_GRAPHS + 1) * EMB_DIM

_sc_mesh = plsc.VectorSubcoreMesh(core_axis_name="c", subcore_axis_name="s")

_sc_params = pltpu.CompilerParams()
if "needs_layout_passes" in pltpu.CompilerParams.__dataclass_fields__:
    _sc_params = dataclasses.replace(_sc_params, needs_layout_passes=False)

_tc_params = pltpu.CompilerParams(vmem_limit_bytes=60 * 1024 * 1024)


def _make_agg_body(cgn, egn):
    ept = EPAD // egn
    nch = ept // ECH
    unroll = 4

    def body(hp_hbm, src_hbm, dst_hbm, out_hbm, utab_v, acc_v, sidx_v,
             didx_v, sem_s0, sem_s1, sem_d0, sem_d1):
        c = lax.axis_index("c")
        s = lax.axis_index("s")
        w = c * NS + s
        cg = w % cgn
        eg = w // cgn
        sem_s = [sem_s0, sem_s1]
        sem_d = [sem_d0, sem_d1]

        pltpu.sync_copy(hp_hbm.at[cg], utab_v)

        seed = jnp.where(eg == 0, jnp.float32(1.0), jnp.float32(0.0))

        @pl.loop(0, TAB // LANES)
        def _(i):
            sl = pl.ds(i * LANES, LANES)
            acc_v[sl] = utab_v[sl] * seed

        planes_u = [utab_v.at[pl.ds(j * PAD_NODES, PAD_NODES)]
                    for j in range(CW)]
        planes_a = [acc_v.at[pl.ds(j * PAD_NODES, PAD_NODES)]
                    for j in range(CW)]

        def copies(k, b):
            base = eg * ept + k * ECH
            return (pltpu.make_async_copy(src_hbm.at[pl.ds(base, ECH)],
                                          sidx_v.at[b], sem_s[b]),
                    pltpu.make_async_copy(dst_hbm.at[pl.ds(base, ECH)],
                                          didx_v.at[b], sem_d[b]))

        def issue(k, b):
            for cp in copies(k, b):
                cp.start()

        def wait(k, b):
            for cp in copies(k, b):
                cp.wait()

        def compute(b):
            @plsc.parallel_loop(0, ECH // LANES, unroll=unroll)
            def _(i):
                sl = pl.ds(i * LANES, LANES)
                sv = sidx_v[b, sl]
                dv = didx_v[b, sl]
                for j in range(CW):
                    vals = plsc.load_gather(planes_u[j], [sv])
                    plsc.addupdate_scatter(planes_a[j], [dv], vals)

        issue(0, 0)

        @pl.loop(0, nch, step=2)
        def _(k):
            issue(k + 1, 1)
            wait(k, 0)
            compute(0)

            @pl.when(k + 2 < nch)
            def _():
                issue(k + 2, 0)

            wait(k + 1, 1)
            compute(1)

        pltpu.sync_copy(acc_v, out_hbm.at[w])

    return body


def _make_sc_agg(cgn, egn):
    return pl.kernel(
        _make_agg_body(cgn, egn),
        out_type=jax.ShapeDtypeStruct((NW, TAB), jnp.float32),
        mesh=_sc_mesh,
        scratch_types=[
            pltpu.VMEM((TAB,), jnp.float32),
            pltpu.VMEM((TAB,), jnp.float32),
            pltpu.VMEM((2, ECH), jnp.int32),
            pltpu.VMEM((2, ECH), jnp.int32),
            pltpu.SemaphoreType.DMA,
            pltpu.SemaphoreType.DMA,
            pltpu.SemaphoreType.DMA,
            pltpu.SemaphoreType.DMA,
        ],
        compiler_params=_sc_params,
    )


@jax.jit
def _sc_agg_l1(h_pt, src, dst):
    return _make_sc_agg(32, 1)(h_pt, src, dst)


@jax.jit
def _sc_agg_l23(h_pt, src, dst):
    return _make_sc_agg(8, 4)(h_pt, src, dst)


def _to_planes(h, cgn):
    return h.T.reshape(cgn, CW, PAD_NODES).reshape(cgn, TAB)


def _from_partials(out, cgn, egn):
    return (out.reshape(egn, cgn, CW, PAD_NODES)
            .transpose(0, 3, 1, 2)
            .reshape(egn, PAD_NODES, cgn * CW))


def _pool_body(m_hbm, b_hbm, out_hbm, rows_v, bid_v, pacc_v):
    c = lax.axis_index("c")
    s = lax.axis_index("s")
    w = c * NS + s
    base = w * POOL_RPT

    pltpu.sync_copy(m_hbm.at[pl.ds(base, POOL_RPT)], rows_v)
    pltpu.sync_copy(b_hbm.at[pl.ds(base, POOL_RPT)], bid_v)

    @pl.loop(0, POOL_TAB // LANES)
    def _(i):
        pacc_v[pl.ds(i * LANES, LANES)] = jnp.zeros((LANES,), jnp.float32)

    colofs = lax.broadcasted_iota(jnp.int32, (LANES,), 0)

    @pl.loop(0, POOL_RPT // LANES)
    def _(g):
        for j in range(LANES):
            r = g * LANES + j
            rep = plsc.load_gather(
                bid_v, [jnp.zeros((LANES,), jnp.int32) + r])
            gbase = rep * EMB_DIM
            for k in range(EMB_DIM // LANES):
                vals = rows_v[r, pl.ds(k * LANES, LANES)]
                plsc.addupdate_scatter(
                    pacc_v, [gbase + (colofs + k * LANES)], vals)

    pltpu.sync_copy(pacc_v, out_hbm.at[w])


@jax.jit
def _sc_pool(m_p, batch_p):
    kern = pl.kernel(
        _pool_body,
        out_type=jax.ShapeDtypeStruct((NW, POOL_TAB), jnp.float32),
        mesh=_sc_mesh,
        scratch_types=[
            pltpu.VMEM((POOL_RPT, EMB_DIM), jnp.float32),
            pltpu.VMEM((POOL_RPT,), jnp.int32),
            pltpu.VMEM((POOL_TAB,), jnp.float32),
        ],
        compiler_params=_sc_params,
    )
    return kern(m_p, batch_p)


def _layer_kernel(m_ref, w1_ref, b1_ref, w2_ref, b2_ref, g_ref, bb_ref,
                  h_ref):
    m = m_ref[0]
    for e in range(1, m_ref.shape[0]):
        m = m + m_ref[e]
    pre = jnp.dot(m, w1_ref[...], preferred_element_type=jnp.float32) \
        + b1_ref[...]
    t = jnp.maximum(pre, 0.0)
    mm = jnp.dot(t, w2_ref[...], preferred_element_type=jnp.float32)
    mm = jnp.maximum(mm + b2_ref[...], 0.0)
    valid = lax.broadcasted_iota(jnp.int32, mm.shape, 0) < N_NODES
    mm = jnp.where(valid, mm, 0.0)
    mean = jnp.sum(mm, axis=0, keepdims=True) * (1.0 / N_NODES)
    d = jnp.where(valid, mm - mean, 0.0)
    var = jnp.sum(d * d, axis=0, keepdims=True) * (1.0 / N_NODES)
    h = d * jax.lax.rsqrt(var + 1e-5) * g_ref[...] + bb_ref[...]
    h_ref[...] = jnp.where(valid, h, 0.0)


@jax.jit
def _tc_layer(m_parts, w1, b1, w2, b2, g, bb):
    n = m_parts.shape[1]
    return pl.pallas_call(
        _layer_kernel,
        out_shape=jax.ShapeDtypeStruct((n, HIDDEN), jnp.float32),
        compiler_params=_tc_params,
    )(m_parts, w1, b1, w2, b2, g, bb)


def _ff_block(h, w1, b1, w2, b2, w3, b3, ws, bs):
    z = jnp.maximum(jnp.dot(h, w1, preferred_element_type=jnp.float32) + b1,
                    0.0)
    z = jnp.maximum(jnp.dot(z, w2, preferred_element_type=jnp.float32) + b2,
                    0.0)
    z = jnp.maximum(jnp.dot(z, w3, preferred_element_type=jnp.float32) + b3,
                    0.0)
    return z + jnp.dot(h, ws, preferred_element_type=jnp.float32) + bs


def _heads_kernel(m_ref, p_ref,
                  lw1, lb1, lw2, lb2, lw3, lb3, lws, lbs,
                  gw1, gb1, gw2, gb2, gw3, gb3, gws, gbs,
                  cw_ref, cb_ref, cl_ref,
                  z_ref, q_ref, g_ref, l_ref):
    y = jnp.sum(p_ref[...], axis=0)[:NUM_GRAPHS]
    g_ref[...] = _ff_block(y, gw1[...], gb1[...], gw2[...], gb2[...],
                           gw3[...], gb3[...], gws[...], gbs[...])
    l_ref[...] = _ff_block(m_ref[...], lw1[...], lb1[...], lw2[...], lb2[...],
                           lw3[...], lb3[...], lws[...], lbs[...])
    z = jnp.dot(y, cw_ref[...], preferred_element_type=jnp.float32) \
        + cb_ref[...]
    z_ref[...] = z
    cl = cl_ref[...]
    diff = z[:, None, :] - cl[None, :, :]
    dist = jnp.sum(diff * diff, axis=2)
    q = 1.0 / (1.0 + dist)
    q_ref[...] = q / jnp.sum(q, axis=1, keepdims=True)


@jax.jit
def _tc_heads(m_p, pooled, ld, gd, cw, cb, cl):
    n = m_p.shape[0]
    return pl.pallas_call(
        _heads_kernel,
        out_shape=(jax.ShapeDtypeStruct((NUM_GRAPHS, HIDDEN), jnp.float32),
                   jax.ShapeDtypeStruct((NUM_GRAPHS, HIDDEN), jnp.float32),
                   jax.ShapeDtypeStruct((NUM_GRAPHS, EMB_DIM), jnp.float32),
                   jax.ShapeDtypeStruct((n, EMB_DIM), jnp.float32)),
        compiler_params=_tc_params,
    )(m_p, pooled,
      ld['W1'], ld['b1'].reshape(1, -1), ld['W2'], ld['b2'].reshape(1, -1),
      ld['W3'], ld['b3'].reshape(1, -1), ld['Ws'], ld['bs'].reshape(1, -1),
      gd['W1'], gd['b1'].reshape(1, -1), gd['W2'], gd['b2'].reshape(1, -1),
      gd['W3'], gd['b3'].reshape(1, -1), gd['Ws'], gd['bs'].reshape(1, -1),
      cw, cb.reshape(1, -1), cl)


def kernel(x, edge_index, batch, num_graphs, params):
    del num_graphs
    pad_e = EPAD - N_EDGES
    src = jnp.concatenate(
        [edge_index[0], jnp.full((pad_e,), N_NODES, jnp.int32)])
    dst = jnp.concatenate(
        [edge_index[1], jnp.full((pad_e,), N_NODES, jnp.int32)])
    x_p = jnp.pad(x, ((0, PAD_NODES - N_NODES), (0, 0)))
    batch_p = jnp.concatenate(
        [batch, jnp.full((POOL_PAD - N_NODES,), NUM_GRAPHS, jnp.int32)])

    gin = params['gin']
    h = x_p
    hs = []
    for l in range(NUM_LAYERS):
        cgn, egn = (32, 1) if l == 0 else (8, 4)
        agg_fn = _sc_agg_l1 if l == 0 else _sc_agg_l23
        out = agg_fn(_to_planes(h, cgn), src, dst)
        m_parts = _from_partials(out, cgn, egn)
        p = gin[l]
        h = _tc_layer(m_parts, p['W1'], p['b1'].reshape(1, -1), p['W2'],
                      p['b2'].reshape(1, -1), p['bn_g'].reshape(1, -1),
                      p['bn_b'].reshape(1, -1))
        hs.append(h)

    m_p = jnp.pad(jnp.concatenate([h[:N_NODES] for h in hs], axis=1),
                  ((0, POOL_PAD - N_NODES), (0, 0)))
    pooled = _sc_pool(m_p, batch_p).reshape(NW, NUM_GRAPHS + 1, EMB_DIM)
    z, q, g_enc, l_enc_p = _tc_heads(
        m_p, pooled, params['local_d'], params['global_d'],
        params['cluster_W'], params['cluster_b'], params['cluster_layer'])
    return z, q, g_enc, l_enc_p[:N_NODES]


# speedup vs baseline: 1.0344x; 1.0232x over previous
import dataclasses
import functools

import jax
import jax.numpy as jnp
from jax import lax
from jax.experimental import pallas as pl
from jax.experimental.pallas import tpu as pltpu
from jax.experimental.pallas import tpu_sc as plsc

N_NODES = 10000
D_FEAT = 128
HIDDEN = 32
NUM_LAYERS = 3
N_EDGES = 320000
NUM_GRAPHS = 64
EMB_DIM = HIDDEN * NUM_LAYERS

NC = 2
NS = 16
NW = NC * NS
LANES = 16

PAD_NODES = 10112
CW = 4
TAB = PAD_NODES * CW
ECH = 8192
EPAD = 327680

POOL_RPT = 320
POOL_PAD = NW * POOL_RPT
POOL_TAB = (NUM_GRAPHS + 1) * EMB_DIM

_sc_mesh = plsc.VectorSubcoreMesh(core_axis_name="c", subcore_axis_name="s")

_sc_params = pltpu.CompilerParams()
if "needs_layout_passes" in pltpu.CompilerParams.__dataclass_fields__:
    _sc_params = dataclasses.replace(_sc_params, needs_layout_passes=False)

_tc_params = pltpu.CompilerParams(vmem_limit_bytes=60 * 1024 * 1024)


def _make_agg_body(cgn, egn):
    ept = EPAD // egn
    nch = ept // ECH
    unroll = 4

    def body(hp_hbm, src_hbm, dst_hbm, out_hbm, utab_v, acc_v, sidx_v,
             didx_v, sem_s0, sem_s1, sem_d0, sem_d1):
        c = lax.axis_index("c")
        s = lax.axis_index("s")
        w = c * NS + s
        cg = w % cgn
        eg = w // cgn
        sem_s = [sem_s0, sem_s1]
        sem_d = [sem_d0, sem_d1]

        planes_u = [utab_v.at[pl.ds(j * PAD_NODES, PAD_NODES)]
                    for j in range(CW)]
        planes_a = [acc_v.at[pl.ds(j * PAD_NODES, PAD_NODES)]
                    for j in range(CW)]

        def copies(k, b):
            base = eg * ept + k * ECH
            return (pltpu.make_async_copy(src_hbm.at[pl.ds(base, ECH)],
                                          sidx_v.at[b], sem_s[b]),
                    pltpu.make_async_copy(dst_hbm.at[pl.ds(base, ECH)],
                                          didx_v.at[b], sem_d[b]))

        def issue(k, b):
            for cp in copies(k, b):
                cp.start()

        def wait(k, b):
            for cp in copies(k, b):
                cp.wait()

        def compute(b):
            @plsc.parallel_loop(0, ECH // LANES, unroll=unroll)
            def _(i):
                sl = pl.ds(i * LANES, LANES)
                sv = sidx_v[b, sl]
                dv = didx_v[b, sl]
                for j in range(CW):
                    vals = plsc.load_gather(planes_u[j], [sv])
                    plsc.addupdate_scatter(planes_a[j], [dv], vals)

        issue(0, 0)
        issue(1, 1)
        pltpu.sync_copy(hp_hbm.at[cg], utab_v)

        seed = jnp.where(eg == 0, jnp.float32(1.0), jnp.float32(0.0))

        @plsc.parallel_loop(0, TAB // LANES, unroll=4)
        def _(i):
            sl = pl.ds(i * LANES, LANES)
            acc_v[sl] = utab_v[sl] * seed

        @pl.loop(0, nch, step=2)
        def _(k):
            wait(k, 0)
            compute(0)

            @pl.when(k + 2 < nch)
            def _():
                issue(k + 2, 0)

            wait(k + 1, 1)
            compute(1)

            @pl.when(k + 3 < nch)
            def _():
                issue(k + 3, 1)

        pltpu.sync_copy(acc_v, out_hbm.at[w])

    return body


def _make_sc_agg(cgn, egn):
    return pl.kernel(
        _make_agg_body(cgn, egn),
        out_type=jax.ShapeDtypeStruct((NW, TAB), jnp.float32),
        mesh=_sc_mesh,
        scratch_types=[
            pltpu.VMEM((TAB,), jnp.float32),
            pltpu.VMEM((TAB,), jnp.float32),
            pltpu.VMEM((2, ECH), jnp.int32),
            pltpu.VMEM((2, ECH), jnp.int32),
            pltpu.SemaphoreType.DMA,
            pltpu.SemaphoreType.DMA,
            pltpu.SemaphoreType.DMA,
            pltpu.SemaphoreType.DMA,
        ],
        compiler_params=_sc_params,
    )


@jax.jit
def _sc_agg_l1(h_pt, src, dst):
    return _make_sc_agg(32, 1)(h_pt, src, dst)


@jax.jit
def _sc_agg_l23(h_pt, src, dst):
    return _make_sc_agg(8, 4)(h_pt, src, dst)


def _to_planes(h, cgn):
    return h.T.reshape(cgn, CW, PAD_NODES).reshape(cgn, TAB)


def _from_partials(out, cgn, egn):
    return (out.reshape(egn, cgn, CW, PAD_NODES)
            .transpose(0, 3, 1, 2)
            .reshape(egn, PAD_NODES, cgn * CW))


def _pool_body(m_hbm, b_hbm, out_hbm, rows_v, bid_v, pacc_v):
    c = lax.axis_index("c")
    s = lax.axis_index("s")
    w = c * NS + s
    base = w * POOL_RPT

    pltpu.sync_copy(m_hbm.at[pl.ds(base, POOL_RPT)], rows_v)
    pltpu.sync_copy(b_hbm.at[pl.ds(base, POOL_RPT)], bid_v)

    @pl.loop(0, POOL_TAB // LANES)
    def _(i):
        pacc_v[pl.ds(i * LANES, LANES)] = jnp.zeros((LANES,), jnp.float32)

    colofs = lax.broadcasted_iota(jnp.int32, (LANES,), 0)

    @pl.loop(0, POOL_RPT // LANES)
    def _(g):
        for j in range(LANES):
            r = g * LANES + j
            rep = plsc.load_gather(
                bid_v, [jnp.zeros((LANES,), jnp.int32) + r])
            gbase = rep * EMB_DIM
            for k in range(EMB_DIM // LANES):
                vals = rows_v[r, pl.ds(k * LANES, LANES)]
                plsc.addupdate_scatter(
                    pacc_v, [gbase + (colofs + k * LANES)], vals)

    pltpu.sync_copy(pacc_v, out_hbm.at[w])


@jax.jit
def _sc_pool(m_p, batch_p):
    kern = pl.kernel(
        _pool_body,
        out_type=jax.ShapeDtypeStruct((NW, POOL_TAB), jnp.float32),
        mesh=_sc_mesh,
        scratch_types=[
            pltpu.VMEM((POOL_RPT, EMB_DIM), jnp.float32),
            pltpu.VMEM((POOL_RPT,), jnp.int32),
            pltpu.VMEM((POOL_TAB,), jnp.float32),
        ],
        compiler_params=_sc_params,
    )
    return kern(m_p, batch_p)


def _layer_kernel(m_ref, w1_ref, b1_ref, w2_ref, b2_ref, g_ref, bb_ref,
                  h_ref):
    m = m_ref[0]
    for e in range(1, m_ref.shape[0]):
        m = m + m_ref[e]
    pre = jnp.dot(m, w1_ref[...], preferred_element_type=jnp.float32) \
        + b1_ref[...]
    t = jnp.maximum(pre, 0.0)
    mm = jnp.dot(t, w2_ref[...], preferred_element_type=jnp.float32)
    mm = jnp.maximum(mm + b2_ref[...], 0.0)
    valid = lax.broadcasted_iota(jnp.int32, mm.shape, 0) < N_NODES
    mm = jnp.where(valid, mm, 0.0)
    mean = jnp.sum(mm, axis=0, keepdims=True) * (1.0 / N_NODES)
    d = jnp.where(valid, mm - mean, 0.0)
    var = jnp.sum(d * d, axis=0, keepdims=True) * (1.0 / N_NODES)
    h = d * jax.lax.rsqrt(var + 1e-5) * g_ref[...] + bb_ref[...]
    h_ref[...] = jnp.where(valid, h, 0.0)


@jax.jit
def _tc_layer(m_parts, w1, b1, w2, b2, g, bb):
    n = m_parts.shape[1]
    return pl.pallas_call(
        _layer_kernel,
        out_shape=jax.ShapeDtypeStruct((n, HIDDEN), jnp.float32),
        compiler_params=_tc_params,
    )(m_parts, w1, b1, w2, b2, g, bb)


def _ff_block(h, w1, b1, w2, b2, w3, b3, ws, bs):
    z = jnp.maximum(jnp.dot(h, w1, preferred_element_type=jnp.float32) + b1,
                    0.0)
    z = jnp.maximum(jnp.dot(z, w2, preferred_element_type=jnp.float32) + b2,
                    0.0)
    z = jnp.maximum(jnp.dot(z, w3, preferred_element_type=jnp.float32) + b3,
                    0.0)
    return z + jnp.dot(h, ws, preferred_element_type=jnp.float32) + bs


def _heads_kernel(m_ref, p_ref,
                  lw1, lb1, lw2, lb2, lw3, lb3, lws, lbs,
                  gw1, gb1, gw2, gb2, gw3, gb3, gws, gbs,
                  cw_ref, cb_ref, cl_ref,
                  z_ref, q_ref, g_ref, l_ref):
    y = jnp.sum(p_ref[...], axis=0)[:NUM_GRAPHS]
    g_ref[...] = _ff_block(y, gw1[...], gb1[...], gw2[...], gb2[...],
                           gw3[...], gb3[...], gws[...], gbs[...])
    l_ref[...] = _ff_block(m_ref[...], lw1[...], lb1[...], lw2[...], lb2[...],
                           lw3[...], lb3[...], lws[...], lbs[...])
    z = jnp.dot(y, cw_ref[...], preferred_element_type=jnp.float32) \
        + cb_ref[...]
    z_ref[...] = z
    cl = cl_ref[...]
    diff = z[:, None, :] - cl[None, :, :]
    dist = jnp.sum(diff * diff, axis=2)
    q = 1.0 / (1.0 + dist)
    q_ref[...] = q / jnp.sum(q, axis=1, keepdims=True)


@jax.jit
def _tc_heads(m_p, pooled, ld, gd, cw, cb, cl):
    n = m_p.shape[0]
    return pl.pallas_call(
        _heads_kernel,
        out_shape=(jax.ShapeDtypeStruct((NUM_GRAPHS, HIDDEN), jnp.float32),
                   jax.ShapeDtypeStruct((NUM_GRAPHS, HIDDEN), jnp.float32),
                   jax.ShapeDtypeStruct((NUM_GRAPHS, EMB_DIM), jnp.float32),
                   jax.ShapeDtypeStruct((n, EMB_DIM), jnp.float32)),
        compiler_params=_tc_params,
    )(m_p, pooled,
      ld['W1'], ld['b1'].reshape(1, -1), ld['W2'], ld['b2'].reshape(1, -1),
      ld['W3'], ld['b3'].reshape(1, -1), ld['Ws'], ld['bs'].reshape(1, -1),
      gd['W1'], gd['b1'].reshape(1, -1), gd['W2'], gd['b2'].reshape(1, -1),
      gd['W3'], gd['b3'].reshape(1, -1), gd['Ws'], gd['bs'].reshape(1, -1),
      cw, cb.reshape(1, -1), cl)


def kernel(x, edge_index, batch, num_graphs, params):
    del num_graphs
    pad_e = EPAD - N_EDGES
    src = jnp.concatenate(
        [edge_index[0], jnp.full((pad_e,), N_NODES, jnp.int32)])
    dst = jnp.concatenate(
        [edge_index[1], jnp.full((pad_e,), N_NODES, jnp.int32)])
    x_p = jnp.pad(x, ((0, PAD_NODES - N_NODES), (0, 0)))
    batch_p = jnp.concatenate(
        [batch, jnp.full((POOL_PAD - N_NODES,), NUM_GRAPHS, jnp.int32)])

    gin = params['gin']
    h = x_p
    hs = []
    for l in range(NUM_LAYERS):
        cgn, egn = (32, 1) if l == 0 else (8, 4)
        agg_fn = _sc_agg_l1 if l == 0 else _sc_agg_l23
        out = agg_fn(_to_planes(h, cgn), src, dst)
        m_parts = _from_partials(out, cgn, egn)
        p = gin[l]
        h = _tc_layer(m_parts, p['W1'], p['b1'].reshape(1, -1), p['W2'],
                      p['b2'].reshape(1, -1), p['bn_g'].reshape(1, -1),
                      p['bn_b'].reshape(1, -1))
        hs.append(h)

    m_p = jnp.pad(jnp.concatenate([h[:N_NODES] for h in hs], axis=1),
                  ((0, POOL_PAD - N_NODES), (0, 0)))
    pooled = _sc_pool(m_p, batch_p).reshape(NW, NUM_GRAPHS + 1, EMB_DIM)
    z, q, g_enc, l_enc_p = _tc_heads(
        m_p, pooled, params['local_d'], params['global_d'],
        params['cluster_W'], params['cluster_b'], params['cluster_layer'])
    return z, q, g_enc, l_enc_p[:N_NODES]


# speedup vs baseline: 1.0591x; 1.0239x over previous
import dataclasses

import jax
import jax.numpy as jnp
from jax import lax
from jax.experimental import pallas as pl
from jax.experimental.pallas import tpu as pltpu
from jax.experimental.pallas import tpu_sc as plsc

N_NODES = 10000
D_FEAT = 128
HIDDEN = 32
NUM_LAYERS = 3
N_EDGES = 320000
NUM_GRAPHS = 64
EMB_DIM = HIDDEN * NUM_LAYERS

NC = 2
NS = 16
NW = NC * NS
LANES = 16

PAD_NODES = 10112
CW = 4
TAB = PAD_NODES * CW
ECH = 8192
EPAD = 327680

POOL_PAD = 10240

_sc_mesh = plsc.VectorSubcoreMesh(core_axis_name="c", subcore_axis_name="s")

_sc_params = pltpu.CompilerParams()
if "needs_layout_passes" in pltpu.CompilerParams.__dataclass_fields__:
    _sc_params = dataclasses.replace(_sc_params, needs_layout_passes=False)

_tc_params = pltpu.CompilerParams(vmem_limit_bytes=60 * 1024 * 1024)


def _make_agg_body(cgn, egn):
    ept = EPAD // egn
    nch = ept // ECH
    unroll = 4

    def body(hp_hbm, src_hbm, dst_hbm, out_hbm, utab_v, acc_v, sidx_v,
             didx_v, sem_s0, sem_s1, sem_d0, sem_d1):
        c = lax.axis_index("c")
        s = lax.axis_index("s")
        w = c * NS + s
        cg = w % cgn
        eg = w // cgn
        sem_s = [sem_s0, sem_s1]
        sem_d = [sem_d0, sem_d1]

        planes_u = [utab_v.at[pl.ds(j * PAD_NODES, PAD_NODES)]
                    for j in range(CW)]
        planes_a = [acc_v.at[pl.ds(j * PAD_NODES, PAD_NODES)]
                    for j in range(CW)]

        def copies(k, b):
            base = eg * ept + k * ECH
            return (pltpu.make_async_copy(src_hbm.at[pl.ds(base, ECH)],
                                          sidx_v.at[b], sem_s[b]),
                    pltpu.make_async_copy(dst_hbm.at[pl.ds(base, ECH)],
                                          didx_v.at[b], sem_d[b]))

        def issue(k, b):
            for cp in copies(k, b):
                cp.start()

        def wait(k, b):
            for cp in copies(k, b):
                cp.wait()

        def compute(b):
            @plsc.parallel_loop(0, ECH // LANES, unroll=unroll)
            def _(i):
                sl = pl.ds(i * LANES, LANES)
                sv = sidx_v[b, sl]
                dv = didx_v[b, sl]
                for j in range(CW):
                    vals = plsc.load_gather(planes_u[j], [sv])
                    plsc.addupdate_scatter(planes_a[j], [dv], vals)

        issue(0, 0)
        issue(1, 1)
        pltpu.sync_copy(hp_hbm.at[cg], utab_v)

        seed = jnp.where(eg == 0, jnp.float32(1.0), jnp.float32(0.0))

        @plsc.parallel_loop(0, TAB // LANES, unroll=4)
        def _(i):
            sl = pl.ds(i * LANES, LANES)
            acc_v[sl] = utab_v[sl] * seed

        @pl.loop(0, nch, step=2)
        def _(k):
            wait(k, 0)
            compute(0)

            @pl.when(k + 2 < nch)
            def _():
                issue(k + 2, 0)

            wait(k + 1, 1)
            compute(1)

            @pl.when(k + 3 < nch)
            def _():
                issue(k + 3, 1)

        pltpu.sync_copy(acc_v, out_hbm.at[w])

    return body


def _make_sc_agg(cgn, egn):
    return pl.kernel(
        _make_agg_body(cgn, egn),
        out_type=jax.ShapeDtypeStruct((NW, TAB), jnp.float32),
        mesh=_sc_mesh,
        scratch_types=[
            pltpu.VMEM((TAB,), jnp.float32),
            pltpu.VMEM((TAB,), jnp.float32),
            pltpu.VMEM((2, ECH), jnp.int32),
            pltpu.VMEM((2, ECH), jnp.int32),
            pltpu.SemaphoreType.DMA,
            pltpu.SemaphoreType.DMA,
            pltpu.SemaphoreType.DMA,
            pltpu.SemaphoreType.DMA,
        ],
        compiler_params=_sc_params,
    )


@jax.jit
def _sc_agg_l1(h_pt, src, dst):
    return _make_sc_agg(32, 1)(h_pt, src, dst)


@jax.jit
def _sc_agg_l23(h_pt, src, dst):
    return _make_sc_agg(8, 4)(h_pt, src, dst)


def _to_planes(h, cgn):
    return h.T.reshape(cgn, CW, PAD_NODES).reshape(cgn, TAB)


def _from_partials(out, cgn, egn):
    return (out.reshape(egn, cgn, CW, PAD_NODES)
            .transpose(0, 3, 1, 2)
            .reshape(egn, PAD_NODES, cgn * CW))


def _layer_kernel(m_ref, w1_ref, b1_ref, w2_ref, b2_ref, g_ref, bb_ref,
                  h_ref):
    m = m_ref[0]
    for e in range(1, m_ref.shape[0]):
        m = m + m_ref[e]
    pre = jnp.dot(m, w1_ref[...], preferred_element_type=jnp.float32) \
        + b1_ref[...]
    t = jnp.maximum(pre, 0.0)
    mm = jnp.dot(t, w2_ref[...], preferred_element_type=jnp.float32)
    mm = jnp.maximum(mm + b2_ref[...], 0.0)
    valid = lax.broadcasted_iota(jnp.int32, mm.shape, 0) < N_NODES
    mm = jnp.where(valid, mm, 0.0)
    mean = jnp.sum(mm, axis=0, keepdims=True) * (1.0 / N_NODES)
    d = jnp.where(valid, mm - mean, 0.0)
    var = jnp.sum(d * d, axis=0, keepdims=True) * (1.0 / N_NODES)
    h = d * jax.lax.rsqrt(var + 1e-5) * g_ref[...] + bb_ref[...]
    h_ref[...] = jnp.where(valid, h, 0.0)


@jax.jit
def _tc_layer(m_parts, w1, b1, w2, b2, g, bb):
    n = m_parts.shape[1]
    return pl.pallas_call(
        _layer_kernel,
        out_shape=jax.ShapeDtypeStruct((n, HIDDEN), jnp.float32),
        compiler_params=_tc_params,
    )(m_parts, w1, b1, w2, b2, g, bb)


def _ff_block(h, w1, b1, w2, b2, w3, b3, ws, bs):
    z = jnp.maximum(jnp.dot(h, w1, preferred_element_type=jnp.float32) + b1,
                    0.0)
    z = jnp.maximum(jnp.dot(z, w2, preferred_element_type=jnp.float32) + b2,
                    0.0)
    z = jnp.maximum(jnp.dot(z, w3, preferred_element_type=jnp.float32) + b3,
                    0.0)
    return z + jnp.dot(h, ws, preferred_element_type=jnp.float32) + bs


def _heads_kernel(m_ref, b_ref,
                  lw1, lb1, lw2, lb2, lw3, lb3, lws, lbs,
                  gw1, gb1, gw2, gb2, gw3, gb3, gws, gbs,
                  cw_ref, cb_ref, cl_ref,
                  z_ref, q_ref, g_ref, l_ref):
    gids = lax.broadcasted_iota(jnp.int32, (NUM_GRAPHS, POOL_PAD), 0)
    onehot = jnp.where(gids == b_ref[...], 1.0, 0.0)
    y = jnp.dot(onehot, m_ref[...], preferred_element_type=jnp.float32,
                precision=lax.Precision.HIGHEST)
    g_ref[...] = _ff_block(y, gw1[...], gb1[...], gw2[...], gb2[...],
                           gw3[...], gb3[...], gws[...], gbs[...])
    l_ref[...] = _ff_block(m_ref[...], lw1[...], lb1[...], lw2[...], lb2[...],
                           lw3[...], lb3[...], lws[...], lbs[...])
    z = jnp.dot(y, cw_ref[...], preferred_element_type=jnp.float32) \
        + cb_ref[...]
    z_ref[...] = z
    cl = cl_ref[...]
    diff = z[:, None, :] - cl[None, :, :]
    dist = jnp.sum(diff * diff, axis=2)
    q = 1.0 / (1.0 + dist)
    q_ref[...] = q / jnp.sum(q, axis=1, keepdims=True)


@jax.jit
def _tc_heads(m_p, batch_row, ld, gd, cw, cb, cl):
    n = m_p.shape[0]
    return pl.pallas_call(
        _heads_kernel,
        out_shape=(jax.ShapeDtypeStruct((NUM_GRAPHS, HIDDEN), jnp.float32),
                   jax.ShapeDtypeStruct((NUM_GRAPHS, HIDDEN), jnp.float32),
                   jax.ShapeDtypeStruct((NUM_GRAPHS, EMB_DIM), jnp.float32),
                   jax.ShapeDtypeStruct((n, EMB_DIM), jnp.float32)),
        compiler_params=_tc_params,
    )(m_p, batch_row,
      ld['W1'], ld['b1'].reshape(1, -1), ld['W2'], ld['b2'].reshape(1, -1),
      ld['W3'], ld['b3'].reshape(1, -1), ld['Ws'], ld['bs'].reshape(1, -1),
      gd['W1'], gd['b1'].reshape(1, -1), gd['W2'], gd['b2'].reshape(1, -1),
      gd['W3'], gd['b3'].reshape(1, -1), gd['Ws'], gd['bs'].reshape(1, -1),
      cw, cb.reshape(1, -1), cl)


def kernel(x, edge_index, batch, num_graphs, params):
    del num_graphs
    pad_e = EPAD - N_EDGES
    src = jnp.concatenate(
        [edge_index[0], jnp.full((pad_e,), N_NODES, jnp.int32)])
    dst = jnp.concatenate(
        [edge_index[1], jnp.full((pad_e,), N_NODES, jnp.int32)])
    x_p = jnp.pad(x, ((0, PAD_NODES - N_NODES), (0, 0)))
    batch_p = jnp.concatenate(
        [batch, jnp.full((POOL_PAD - N_NODES,), NUM_GRAPHS, jnp.int32)])

    gin = params['gin']
    h = x_p
    hs = []
    for l in range(NUM_LAYERS):
        cgn, egn = (32, 1) if l == 0 else (8, 4)
        agg_fn = _sc_agg_l1 if l == 0 else _sc_agg_l23
        out = agg_fn(_to_planes(h, cgn), src, dst)
        m_parts = _from_partials(out, cgn, egn)
        p = gin[l]
        h = _tc_layer(m_parts, p['W1'], p['b1'].reshape(1, -1), p['W2'],
                      p['b2'].reshape(1, -1), p['bn_g'].reshape(1, -1),
                      p['bn_b'].reshape(1, -1))
        hs.append(h)

    m_p = jnp.pad(jnp.concatenate([h[:N_NODES] for h in hs], axis=1),
                  ((0, POOL_PAD - N_NODES), (0, 0)))
    z, q, g_enc, l_enc_p = _tc_heads(
        m_p, batch_p.reshape(1, POOL_PAD), params['local_d'],
        params['global_d'], params['cluster_W'], params['cluster_b'],
        params['cluster_layer'])
    return z, q, g_enc, l_enc_p[:N_NODES]
